# Initial kernel scaffold; baseline (speedup 1.0000x reference)
import jax
import jax.numpy as jnp
from jax.experimental import pallas as pl


def kernel(x, batch, fil_W, fil_b, tri_t, gauss_mu, gauss_sigma, line_W, line_b, rh_c, rh_r, out_W, out_b):
    raise NotImplementedError("write your pallas kernel here")


# speedup vs baseline: 8.5982x; 8.5982x over previous
import functools

import jax
import jax.numpy as jnp
from jax.experimental import pallas as pl
from jax.experimental.pallas import tpu as pltpu

_G = 512
_F = 8
_D = 256
_BLK = 512
_NEG = -3.4e38


def _fv_segmax_body(x_ref, batch_ref, filw_ref, filb_ref, fv_ref, segmax_ref):
    i = pl.program_id(0)
    x = x_ref[...]
    fv = jnp.dot(x, filw_ref[...], preferred_element_type=jnp.float32)
    fv = fv + filb_ref[...]
    fv_ref[...] = fv

    b = batch_ref[...]
    g = jax.lax.broadcasted_iota(jnp.int32, (_BLK, _G), 1)
    mask = b == g

    @pl.when(i == 0)
    def _():
        segmax_ref[...] = jnp.full((_F, _G), _NEG, jnp.float32)

    rows = []
    for f in range(_F):
        vals = jnp.where(mask, fv[:, f:f + 1], _NEG)
        rows.append(jnp.max(vals, axis=0, keepdims=True))
    upd = jnp.concatenate(rows, axis=0)
    segmax_ref[...] = jnp.maximum(segmax_ref[...], upd)


def _out_body(x_ref, batch_ref, fv_ref, segmax_ref, w1_ref, w2_ref, outb_ref,
              tri_t_ref, gmu_ref, gsig_ref, lw_ref, lb_ref, rhc_ref, rhr_ref,
              out_ref):
    b = batch_ref[...]
    g = jax.lax.broadcasted_iota(jnp.int32, (_BLK, _G), 1)
    onehot = (b == g).astype(jnp.float32)
    death = jax.lax.dot_general(
        onehot, segmax_ref[...],
        dimension_numbers=(((1,), (1,)), ((), ())),
        preferred_element_type=jnp.float32)
    birth = fv_ref[...]

    cs = []
    for k in range(3):
        cs.append(jax.nn.relu(death - jnp.abs(tri_t_ref[k] - birth)))
    for k in range(3):
        d2 = (birth - gmu_ref[k, 0]) ** 2 + (death - gmu_ref[k, 1]) ** 2
        cs.append(jnp.exp(-d2 / (2.0 * gsig_ref[k] ** 2)))
    for k in range(3):
        cs.append(birth * lw_ref[0, k] + death * lw_ref[1, k] + lb_ref[k])
    r_abs = jnp.abs(rhr_ref[0])
    for k in range(3):
        l1 = jnp.abs(birth - rhc_ref[k, 0]) + jnp.abs(death - rhc_ref[k, 1])
        cs.append(1.0 / (1.0 + l1) - 1.0 / (1.0 + jnp.abs(r_abs - l1)))
    coord = jnp.concatenate(cs, axis=1)

    out = jnp.dot(x_ref[...], w1_ref[...], preferred_element_type=jnp.float32)
    out = out + jnp.dot(coord, w2_ref[...], preferred_element_type=jnp.float32)
    out_ref[...] = out + outb_ref[...]


def kernel(x, batch, fil_W, fil_b, tri_t, gauss_mu, gauss_sigma,
           line_W, line_b, rh_c, rh_r, out_W, out_b):
    n = x.shape[0]
    nblk = (n + _BLK - 1) // _BLK
    n_pad = nblk * _BLK
    x_p = jnp.pad(x, ((0, n_pad - n), (0, 0)))
    batch_p = jnp.pad(batch, (0, n_pad - n), constant_values=_G)
    batch_p = batch_p.reshape(n_pad, 1)

    fv, segmax = pl.pallas_call(
        _fv_segmax_body,
        grid=(nblk,),
        in_specs=[
            pl.BlockSpec((_BLK, _D), lambda i: (i, 0)),
            pl.BlockSpec((_BLK, 1), lambda i: (i, 0)),
            pl.BlockSpec((_D, _F), lambda i: (0, 0)),
            pl.BlockSpec((1, _F), lambda i: (0, 0)),
        ],
        out_specs=[
            pl.BlockSpec((_BLK, _F), lambda i: (i, 0)),
            pl.BlockSpec((_F, _G), lambda i: (0, 0)),
        ],
        out_shape=[
            jax.ShapeDtypeStruct((n_pad, _F), jnp.float32),
            jax.ShapeDtypeStruct((_F, _G), jnp.float32),
        ],
    )(x_p, batch_p, fil_W, fil_b.reshape(1, _F))

    w2 = out_W[_D:].reshape(_F, 12, _D).transpose(1, 0, 2).reshape(12 * _F, _D)

    smem = pl.BlockSpec(memory_space=pltpu.SMEM)
    out_p = pl.pallas_call(
        _out_body,
        grid=(nblk,),
        in_specs=[
            pl.BlockSpec((_BLK, _D), lambda i: (i, 0)),
            pl.BlockSpec((_BLK, 1), lambda i: (i, 0)),
            pl.BlockSpec((_BLK, _F), lambda i: (i, 0)),
            pl.BlockSpec((_F, _G), lambda i: (0, 0)),
            pl.BlockSpec((_D, _D), lambda i: (0, 0)),
            pl.BlockSpec((12 * _F, _D), lambda i: (0, 0)),
            pl.BlockSpec((1, _D), lambda i: (0, 0)),
            smem, smem, smem, smem, smem, smem, smem,
        ],
        out_specs=pl.BlockSpec((_BLK, _D), lambda i: (i, 0)),
        out_shape=jax.ShapeDtypeStruct((n_pad, _D), jnp.float32),
    )(x_p, batch_p, fv, segmax, out_W[:_D], w2, out_b.reshape(1, _D),
      tri_t, gauss_mu, gauss_sigma, line_W, line_b, rh_c, rh_r)

    return out_p[:n]
